# baseline (speedup 1.0000x reference)
import functools

import jax
import jax.numpy as jnp
from jax.experimental import pallas as pl
from jax.experimental.pallas import tpu as pltpu

_TILE = 1024
_NCORE = 2


def _fused_body(starts_ref, ends_ref, x_ref, W1_ref, b1_ref, W2_ref, b2_ref,
                W3_ref, b3_ref, out_ref, acc_ref, *, tile, inner, nseg):
    c = pl.program_id(0)
    i = pl.program_id(1)

    @pl.when(i == 0)
    def _init():
        acc_ref[...] = jnp.zeros_like(acc_ref)

    x = x_ref[...]
    h = jnp.maximum(jnp.dot(x, W1_ref[...]) + b1_ref[...], 0.0)
    h = jnp.maximum(jnp.dot(h, W2_ref[...]) + b2_ref[...], 0.0)

    rows = ((c * inner + i) * tile
            + jax.lax.broadcasted_iota(jnp.int32, (tile, nseg), 0))
    starts = starts_ref[...]
    ends = ends_ref[...]
    onehot = ((rows >= starts) & (rows < ends)).astype(jnp.float32)
    acc_ref[...] += jax.lax.dot_general(
        onehot, h, dimension_numbers=(((0,), (0,)), ((), ())))

    @pl.when(i == inner - 1)
    def _finish():
        part = jnp.dot(acc_ref[...], W3_ref[...])
        counts = (ends - starts).astype(jnp.float32).reshape(nseg, 1)
        bias = jnp.where(c == 0, counts * b3_ref[...], 0.0)
        out_ref[0] = part + bias


def kernel(flat, cu_seqlens, W1, b1, W2, b2, W3, b3):
    T, D = flat.shape
    H = W1.shape[1]
    O = W3.shape[1]
    nseg = cu_seqlens.shape[0] - 1
    starts = cu_seqlens[:-1].reshape(1, nseg)
    ends = cu_seqlens[1:].reshape(1, nseg)
    inner = T // _TILE // _NCORE
    body = functools.partial(_fused_body, tile=_TILE, inner=inner, nseg=nseg)
    parts = pl.pallas_call(
        body,
        grid=(_NCORE, inner),
        in_specs=[
            pl.BlockSpec((1, nseg), lambda c, i: (0, 0)),
            pl.BlockSpec((1, nseg), lambda c, i: (0, 0)),
            pl.BlockSpec((_TILE, D), lambda c, i: (c * inner + i, 0)),
            pl.BlockSpec((D, H), lambda c, i: (0, 0)),
            pl.BlockSpec((1, H), lambda c, i: (0, 0)),
            pl.BlockSpec((H, H), lambda c, i: (0, 0)),
            pl.BlockSpec((1, H), lambda c, i: (0, 0)),
            pl.BlockSpec((H, O), lambda c, i: (0, 0)),
            pl.BlockSpec((1, O), lambda c, i: (0, 0)),
        ],
        out_specs=pl.BlockSpec((1, nseg, O), lambda c, i: (c, 0, 0)),
        out_shape=jax.ShapeDtypeStruct((_NCORE, nseg, O), jnp.float32),
        scratch_shapes=[pltpu.VMEM((nseg, H), jnp.float32)],
        compiler_params=pltpu.CompilerParams(
            dimension_semantics=("parallel", "arbitrary")),
    )(starts, ends, flat, W1, b1.reshape(1, H), W2, b2.reshape(1, H),
      W3, b3.reshape(1, O))
    return parts.sum(axis=0)
